# Initial kernel scaffold; baseline (speedup 1.0000x reference)
import jax
import jax.numpy as jnp
from jax.experimental import pallas as pl


def kernel(x, W_v_alpha, W_j_alpha, W_v_beta, W_d_beta, W_j_beta):
    raise NotImplementedError("write your pallas kernel here")


# speedup vs baseline: 3.3102x; 3.3102x over previous
import functools

import jax
import jax.numpy as jnp
from jax import lax
from jax.experimental import pallas as pl
from jax.experimental.pallas import tpu as pltpu
from jax.experimental.pallas import tpu_sc as plsc

VDJ_DIM = 64
BATCH = 16384
NUM_TABLES = 5

_NC = 2
_NS = 16
_NW = _NC * _NS
_BPW = BATCH // _NW
_CHUNK = 128
_NCHUNK = _BPW // _CHUNK


def _gather_body(xt_hbm, w0, w1, w2, w3, w4,
                 o0, o1, o2, o3, o4,
                 idx_v, rows_v, sem):
    wid = lax.axis_index("s") * _NC + lax.axis_index("c")
    base = wid * _BPW
    tables = (w0, w1, w2, w3, w4)
    outs = (o0, o1, o2, o3, o4)
    for t in range(NUM_TABLES):
        pltpu.sync_copy(xt_hbm.at[t, wid], idx_v)
        copies = []
        for j in range(_NCHUNK):
            copies.append(
                pltpu.async_copy(
                    tables[t].at[idx_v.at[j]],
                    rows_v.at[pl.ds(j * _CHUNK, _CHUNK)],
                    sem,
                )
            )
        for c in copies:
            c.wait()
        pltpu.sync_copy(rows_v, outs[t].at[pl.ds(base, _BPW)])


@jax.jit
def _vdj_gather(xt, w0, w1, w2, w3, w4):
    kern = pl.kernel(
        _gather_body,
        out_type=tuple(
            jax.ShapeDtypeStruct((BATCH, VDJ_DIM), jnp.float32)
            for _ in range(NUM_TABLES)
        ),
        mesh=plsc.VectorSubcoreMesh(core_axis_name="c", subcore_axis_name="s"),
        scratch_types=[
            pltpu.VMEM((_NCHUNK, _CHUNK), jnp.int32),
            pltpu.VMEM((_BPW, VDJ_DIM), jnp.float32),
            pltpu.SemaphoreType.DMA,
        ],
        compiler_params=pltpu.CompilerParams(use_tc_tiling_on_sc=False),
    )
    return kern(xt, w0, w1, w2, w3, w4)


def kernel(x, W_v_alpha, W_j_alpha, W_v_beta, W_d_beta, W_j_beta):
    xt = x.astype(jnp.int32).T.reshape(NUM_TABLES, _NW, _NCHUNK, _CHUNK)
    return _vdj_gather(xt, W_v_alpha, W_j_alpha, W_v_beta, W_d_beta, W_j_beta)
